# speedup vs baseline: 1.1684x; 1.1684x over previous
import jax
import jax.numpy as jnp
from jax.experimental import pallas as pl

_N, _D, _E, _G, _DF = 2048, 768, 8, 2, 1536
_EG = _E // _G
_TILE = 256


def _head_fold_body(w2e_ref, wr_ref, out_ref):
    out_ref[...] = jnp.dot(w2e_ref[0], wr_ref[...],
                           preferred_element_type=jnp.float32)[None]


def _moe_body(x_ref, wg_ref, we_ref, w1_ref, b1_ref, vc_ref, brisk_ref,
              out_ref):
    xt = x_ref[...]
    gl = jnp.dot(xt, wg_ref[...])
    el = jnp.dot(xt, we_ref[...])
    pg = jax.nn.softmax(gl, axis=-1)
    pe = jax.nn.softmax(el.reshape(_TILE, _G, _EG), axis=-1)
    probs = (pg[:, :, None] * pe).reshape(_TILE, _E)

    iota = jax.lax.broadcasted_iota(jnp.int32, (_TILE, _E), 1)
    v1 = jnp.max(probs, axis=1, keepdims=True)
    i1 = jnp.argmax(probs, axis=1)[:, None]
    m1 = iota == i1
    masked = jnp.where(m1, -jnp.inf, probs)
    v2 = jnp.max(masked, axis=1, keepdims=True)
    i2 = jnp.argmax(masked, axis=1)[:, None]
    m2 = iota == i2
    denom = v1 + v2 + 1e-9
    gates = jnp.where(m1, v1 / denom, 0.0) + jnp.where(m2, v2 / denom, 0.0)

    vc = vc_ref[...]
    acc = jnp.zeros((_TILE, 1), jnp.float32)
    for e in range(_E):
        h = jnp.dot(xt, w1_ref[e]) + b1_ref[e][None]
        h = jax.nn.gelu(h)
        s = jnp.dot(h, vc[e, :_DF])
        acc = acc + gates[:, e][:, None] * (s + vc[e, _DF])
    out_ref[...] = acc + brisk_ref[0, 0]


@jax.jit
def kernel(x, Wg_group, Wg_expert, W1, b1, W2, b2, W_risk, b_risk):
    w2ext = jnp.concatenate([W2, b2[:, None, :]], axis=1)
    vc = pl.pallas_call(
        _head_fold_body,
        grid=(_E,),
        in_specs=[
            pl.BlockSpec((1, _DF + 1, _D), lambda e: (e, 0, 0)),
            pl.BlockSpec((_D, 1), lambda e: (0, 0)),
        ],
        out_specs=pl.BlockSpec((1, _DF + 1, 1), lambda e: (e, 0, 0)),
        out_shape=jax.ShapeDtypeStruct((_E, _DF + 1, 1), jnp.float32),
    )(w2ext, W_risk)

    risk = pl.pallas_call(
        _moe_body,
        grid=(_N // _TILE,),
        in_specs=[
            pl.BlockSpec((_TILE, _D), lambda i: (i, 0)),
            pl.BlockSpec((_D, _G), lambda i: (0, 0)),
            pl.BlockSpec((_D, _E), lambda i: (0, 0)),
            pl.BlockSpec((_E, _D, _DF), lambda i: (0, 0, 0)),
            pl.BlockSpec((_E, _DF), lambda i: (0, 0)),
            pl.BlockSpec((_E, _DF + 1, 1), lambda i: (0, 0, 0)),
            pl.BlockSpec((1, 1), lambda i: (0, 0)),
        ],
        out_specs=pl.BlockSpec((_TILE, 1), lambda i: (i, 0)),
        out_shape=jax.ShapeDtypeStruct((_N, 1), jnp.float32),
    )(x, Wg_group, Wg_expert, W1, b1, vc, b_risk.reshape(1, 1))
    return risk[:, 0]


# speedup vs baseline: 1.4002x; 1.1984x over previous
import jax
import jax.numpy as jnp
from jax.experimental import pallas as pl

_N, _D, _E, _G, _DF = 2048, 768, 8, 2, 1536
_EG = _E // _G
_TILE = 256


def _head_fold_body(w2e_ref, b2e_ref, wr_ref, v_ref, c_ref):
    wr = wr_ref[...]
    v_ref[...] = jnp.dot(w2e_ref[0], wr,
                         preferred_element_type=jnp.float32)[None]
    c_ref[...] = jnp.dot(b2e_ref[0], wr,
                         preferred_element_type=jnp.float32)[None]


def _moe_body(x_ref, wg_ref, we_ref, w1_ref, b1_ref, v_ref, c_ref, brisk_ref,
              out_ref):
    xt = x_ref[...]
    gl = jnp.dot(xt, wg_ref[...])
    el = jnp.dot(xt, we_ref[...])
    pg = jax.nn.softmax(gl, axis=-1)
    pe = jax.nn.softmax(el.reshape(_TILE, _G, _EG), axis=-1)
    probs = (pg[:, :, None] * pe).reshape(_TILE, _E)

    iota = jax.lax.broadcasted_iota(jnp.int32, (_TILE, _E), 1)
    v1 = jnp.max(probs, axis=1, keepdims=True)
    i1 = jnp.argmax(probs, axis=1)[:, None]
    m1 = iota == i1
    masked = jnp.where(m1, -jnp.inf, probs)
    v2 = jnp.max(masked, axis=1, keepdims=True)
    i2 = jnp.argmax(masked, axis=1)[:, None]
    m2 = iota == i2
    denom = v1 + v2 + 1e-9
    gates = jnp.where(m1, v1 / denom, 0.0) + jnp.where(m2, v2 / denom, 0.0)

    acc = jnp.zeros((_TILE, 1), jnp.float32)
    for e in range(_E):
        h = jnp.dot(xt, w1_ref[e]) + b1_ref[e][None]
        h = jax.nn.gelu(h)
        s = jnp.dot(h, v_ref[e])
        acc = acc + gates[:, e][:, None] * (s + c_ref[e, 0, 0])
    out_ref[...] = acc + brisk_ref[0, 0]


@jax.jit
def kernel(x, Wg_group, Wg_expert, W1, b1, W2, b2, W_risk, b_risk):
    v, c = pl.pallas_call(
        _head_fold_body,
        grid=(_E,),
        in_specs=[
            pl.BlockSpec((1, _DF, _D), lambda e: (e, 0, 0)),
            pl.BlockSpec((1, 1, _D), lambda e: (e, 0, 0)),
            pl.BlockSpec((_D, 1), lambda e: (0, 0)),
        ],
        out_specs=[
            pl.BlockSpec((1, _DF, 1), lambda e: (e, 0, 0)),
            pl.BlockSpec((1, 1, 1), lambda e: (e, 0, 0)),
        ],
        out_shape=[
            jax.ShapeDtypeStruct((_E, _DF, 1), jnp.float32),
            jax.ShapeDtypeStruct((_E, 1, 1), jnp.float32),
        ],
    )(W2, b2[:, None, :], W_risk)

    risk = pl.pallas_call(
        _moe_body,
        grid=(_N // _TILE,),
        in_specs=[
            pl.BlockSpec((_TILE, _D), lambda i: (i, 0)),
            pl.BlockSpec((_D, _G), lambda i: (0, 0)),
            pl.BlockSpec((_D, _E), lambda i: (0, 0)),
            pl.BlockSpec((_E, _D, _DF), lambda i: (0, 0, 0)),
            pl.BlockSpec((_E, _DF), lambda i: (0, 0)),
            pl.BlockSpec((_E, _DF, 1), lambda i: (0, 0, 0)),
            pl.BlockSpec((_E, 1, 1), lambda i: (0, 0, 0)),
            pl.BlockSpec((1, 1), lambda i: (0, 0)),
        ],
        out_specs=pl.BlockSpec((_TILE, 1), lambda i: (i, 0)),
        out_shape=jax.ShapeDtypeStruct((_N, 1), jnp.float32),
    )(x, Wg_group, Wg_expert, W1, b1, v, c, b_risk.reshape(1, 1))
    return risk[:, 0]
